# baseline (speedup 1.0000x reference)
import functools

import jax
import jax.numpy as jnp
from jax import lax
from jax.experimental import pallas as pl
from jax.experimental.pallas import tpu as pltpu
from jax.experimental.pallas import tpu_sc as plsc

SIZE_DATASET = 100000
NUM_BONES = 32
BATCH = 4096
D = NUM_BONES * 4

NC = 2
NS = 16
NW = NC * NS
B_PER_W = BATCH // NW
CHUNKS = B_PER_W * D // (4 * 16)


NBLK = 4
ROWS_BLK = B_PER_W // NBLK


def _sc_body(idx_hbm, table_hbm, out_hbm, idx_v, rows_v, sem):
    wid = lax.axis_index("s") * NC + lax.axis_index("c")
    base = wid * B_PER_W
    pltpu.sync_copy(idx_hbm.at[pl.ds(base, B_PER_W)], idx_v)
    pltpu.async_copy(table_hbm.at[idx_v], rows_v, sem).wait()

    lanes = lax.iota(jnp.int32, 16)

    def chunk(i):
        row = i >> 1
        row_v = jnp.full((16,), row, jnp.int32)
        c0 = (i & 1) * 64 + lanes * 4
        c1 = c0 + 1
        c2 = c0 + 2
        c3 = c0 + 3
        qx = plsc.load_gather(rows_v, [row_v, c0])
        qy = plsc.load_gather(rows_v, [row_v, c1])
        qz = plsc.load_gather(rows_v, [row_v, c2])
        qw = plsc.load_gather(rows_v, [row_v, c3])
        ss = jnp.maximum(qx * qx + qy * qy + qz * qz + qw * qw, 1e-24)
        yi = 0x5F3759DF - (lax.bitcast_convert_type(ss, jnp.int32) >> 1)
        y = lax.bitcast_convert_type(yi, jnp.float32)
        y = y * (1.5 - (ss * 0.5) * y * y)
        plsc.store_scatter(rows_v, [row_v, c0], qx * y)
        plsc.store_scatter(rows_v, [row_v, c1], qy * y)
        plsc.store_scatter(rows_v, [row_v, c2], qz * y)
        plsc.store_scatter(rows_v, [row_v, c3], qw * y)

    plsc.parallel_loop(0, CHUNKS, 1, unroll=1)(chunk)
    pltpu.sync_copy(rows_v, out_hbm.at[pl.ds(base, B_PER_W)])


@jax.jit
def _run(indices, table):
    mesh = plsc.VectorSubcoreMesh(core_axis_name="c", subcore_axis_name="s")
    out = pl.kernel(
        _sc_body,
        mesh=mesh,
        out_type=jax.ShapeDtypeStruct((BATCH, D), jnp.float32),
        scratch_types=[
            pltpu.VMEM((B_PER_W,), jnp.int32),
            pltpu.VMEM((B_PER_W, D), jnp.float32),
            pltpu.SemaphoreType.DMA,
        ],
        compiler_params=pltpu.CompilerParams(
            needs_layout_passes=False, use_tc_tiling_on_sc=False
        ),
    )(indices, table)
    return out.reshape(BATCH, NUM_BONES, 4)


def kernel(indices, table):
    return _run(indices.astype(jnp.int32), table)


# speedup vs baseline: 1.0241x; 1.0241x over previous
import functools

import jax
import jax.numpy as jnp
from jax import lax
from jax.experimental import pallas as pl
from jax.experimental.pallas import tpu as pltpu
from jax.experimental.pallas import tpu_sc as plsc

SIZE_DATASET = 100000
NUM_BONES = 32
BATCH = 4096
D = NUM_BONES * 4

NC = 2
NS = 16
NW = NC * NS
B_PER_W = BATCH // NW
CHUNKS = B_PER_W * D // (4 * 16)


NBLK = 4
ROWS_BLK = B_PER_W // NBLK


HALF = B_PER_W // 2


def _sc_body(idx_hbm, table_hbm, out_hbm, idx_v, rows_v, g0, g1, ws):
    wid = lax.axis_index("s") * NC + lax.axis_index("c")
    base = wid * B_PER_W
    pltpu.sync_copy(idx_hbm.at[pl.ds(base, B_PER_W)], idx_v)
    pltpu.async_copy(
        table_hbm.at[idx_v.at[pl.ds(0, HALF)]],
        rows_v.at[pl.ds(0, HALF)],
        g0,
    )
    pltpu.async_copy(
        table_hbm.at[idx_v.at[pl.ds(HALF, HALF)]],
        rows_v.at[pl.ds(HALF, HALF)],
        g1,
    )

    lanes = lax.iota(jnp.int32, 16)

    def chunk(i):
        row = i >> 1
        row_v = jnp.full((16,), row, jnp.int32)
        c0 = (i & 1) * 64 + lanes * 4
        c1 = c0 + 1
        c2 = c0 + 2
        c3 = c0 + 3
        qx = plsc.load_gather(rows_v, [row_v, c0])
        qy = plsc.load_gather(rows_v, [row_v, c1])
        qz = plsc.load_gather(rows_v, [row_v, c2])
        qw = plsc.load_gather(rows_v, [row_v, c3])
        ss = jnp.maximum(qx * qx + qy * qy + qz * qz + qw * qw, 1e-24)
        yi = 0x5F3759DF - (lax.bitcast_convert_type(ss, jnp.int32) >> 1)
        y = lax.bitcast_convert_type(yi, jnp.float32)
        y = y * (1.5 - (ss * 0.5) * y * y)
        plsc.store_scatter(rows_v, [row_v, c0], qx * y)
        plsc.store_scatter(rows_v, [row_v, c1], qy * y)
        plsc.store_scatter(rows_v, [row_v, c2], qz * y)
        plsc.store_scatter(rows_v, [row_v, c3], qw * y)

    for k, g in ((0, g0), (1, g1)):
        pltpu.make_async_copy(
            table_hbm.at[idx_v.at[pl.ds(0, HALF)]],
            rows_v.at[pl.ds(0, HALF)],
            g,
        ).wait()
        plsc.parallel_loop(k * CHUNKS // 2, (k + 1) * CHUNKS // 2, 1, unroll=2)(
            chunk
        )
        pltpu.async_copy(
            rows_v.at[pl.ds(k * HALF, HALF)],
            out_hbm.at[pl.ds(base + k * HALF, HALF)],
            ws,
        )
    pltpu.make_async_copy(rows_v, out_hbm.at[pl.ds(base, B_PER_W)], ws).wait()


@jax.jit
def _run(indices, table):
    mesh = plsc.VectorSubcoreMesh(core_axis_name="c", subcore_axis_name="s")
    out = pl.kernel(
        _sc_body,
        mesh=mesh,
        out_type=jax.ShapeDtypeStruct((BATCH, D), jnp.float32),
        scratch_types=[
            pltpu.VMEM((B_PER_W,), jnp.int32),
            pltpu.VMEM((B_PER_W, D), jnp.float32),
            pltpu.SemaphoreType.DMA,
            pltpu.SemaphoreType.DMA,
            pltpu.SemaphoreType.DMA,
        ],
        compiler_params=pltpu.CompilerParams(
            needs_layout_passes=False, use_tc_tiling_on_sc=False
        ),
    )(indices, table)
    return out.reshape(BATCH, NUM_BONES, 4)


def kernel(indices, table):
    return _run(indices.astype(jnp.int32), table)
